# Initial kernel scaffold; baseline (speedup 1.0000x reference)
import functools

import jax
import jax.numpy as jnp
from jax import lax
from jax.experimental import pallas as pl
from jax.experimental.pallas import tpu as pltpu
from jax.experimental.pallas import tpu_sc as plsc

NUM_CAT = 3
CAT_VOCAB = 100000
NUM_HASH = 3
HASH_BINS = 100000
NUM_CONT_EMB = 2
CONT_BINS = 1000
NUM_CONT_NORM = 2
EMB = 32
BATCH = 16384
OUT_COLS = NUM_CAT * EMB + NUM_HASH * EMB + NUM_CONT_EMB * EMB + NUM_CONT_NORM

_info = plsc.get_sparse_core_info()
NC = _info.num_cores
NS = _info.num_subcores
L = _info.num_lanes
NW = NC * NS
CHUNK = BATCH // NW

BPAD = 1024


def _tower_body(cat_idx_t, hash_idx_t, cont_vals_t, norm_flat,
                cat_tab, hash_tab, cont_tab, bpad, sparams,
                out,
                idxb, rowsb, valsb, bndb, normb, prmb, sem):
    wid = lax.axis_index("s") * NC + lax.axis_index("c")
    base = wid * CHUNK

    pltpu.sync_copy(sparams, prmb)

    for t in range(NUM_CAT):
        pltpu.sync_copy(cat_idx_t.at[t, pl.ds(base, CHUNK)], idxb)
        pltpu.async_copy(cat_tab.at[t].at[idxb], rowsb, sem).wait()
        pltpu.sync_copy(rowsb, out.at[pl.ds(base, CHUNK), pl.ds(t * EMB, EMB)])
    for t in range(NUM_HASH):
        pltpu.sync_copy(hash_idx_t.at[t, pl.ds(base, CHUNK)], idxb)
        pltpu.async_copy(hash_tab.at[t].at[idxb], rowsb, sem).wait()
        pltpu.sync_copy(
            rowsb, out.at[pl.ds(base, CHUNK), pl.ds((NUM_CAT + t) * EMB, EMB)])

    for t in range(NUM_CONT_EMB):
        pltpu.sync_copy(cont_vals_t.at[t, pl.ds(base, CHUNK)], valsb)
        pltpu.sync_copy(bpad.at[t], bndb)
        b0 = prmb[t]
        ist = prmb[2 + t]
        for i in range(CHUNK // L):
            x = valsb[pl.ds(i * L, L)]
            est = (x - b0) * ist
            est = jnp.minimum(jnp.maximum(est, -2.0), float(CONT_BINS + 1))
            c = est.astype(jnp.int32) + 1
            c = jnp.minimum(jnp.maximum(c, 0), CONT_BINS)
            for _ in range(2):
                lo = plsc.load_gather(bndb, [c])
                hi = plsc.load_gather(bndb, [c + 1])
                c = c - (x < lo).astype(jnp.int32) + (x >= hi).astype(jnp.int32)
            idxb[pl.ds(i * L, L)] = c
        pltpu.async_copy(cont_tab.at[t].at[idxb], rowsb, sem).wait()
        pltpu.sync_copy(
            rowsb,
            out.at[pl.ds(base, CHUNK),
                   pl.ds((NUM_CAT + NUM_HASH + t) * EMB, EMB)])

    pltpu.sync_copy(norm_flat.at[pl.ds(base * NUM_CONT_NORM,
                                       CHUNK * NUM_CONT_NORM)], valsb)
    mean_pat = prmb[4]
    std_pat = prmb[5]
    lane = lax.iota(jnp.int32, L)
    col_idx = lane & 1
    for i in range(CHUNK * NUM_CONT_NORM // L):
        x = valsb[pl.ds(i * L, L)]
        y = (x - mean_pat) / std_pat
        row_idx = (lane + i * L) >> 1
        plsc.store_scatter(normb, [row_idx, col_idx], y)
    pltpu.sync_copy(
        normb,
        out.at[pl.ds(base, CHUNK),
               pl.ds((NUM_CAT + NUM_HASH + NUM_CONT_EMB) * EMB,
                     NUM_CONT_NORM)])


_tower = functools.partial(
    pl.kernel,
    mesh=plsc.VectorSubcoreMesh(core_axis_name="c", subcore_axis_name="s"),
    out_type=jax.ShapeDtypeStruct((BATCH, OUT_COLS), jnp.float32),
    scratch_types=[
        pltpu.VMEM((CHUNK,), jnp.int32),
        pltpu.VMEM((CHUNK, EMB), jnp.float32),
        pltpu.VMEM((CHUNK * NUM_CONT_NORM,), jnp.float32),
        pltpu.VMEM((BPAD,), jnp.float32),
        pltpu.VMEM((CHUNK, NUM_CONT_NORM), jnp.float32),
        pltpu.VMEM((6, L), jnp.float32),
        pltpu.SemaphoreType.DMA,
    ],
)(_tower_body)


def kernel(cat_idx, hash_idx, cont_embed_vals, cont_norm_vals, cat_tables,
           hash_tables, cont_tables, cont_boundaries, norm_mean, norm_std):
    cat_idx_t = cat_idx.astype(jnp.int32).T
    hash_idx_t = hash_idx.astype(jnp.int32).T
    cont_vals_t = cont_embed_vals.T
    norm_flat = cont_norm_vals.reshape(-1)

    neg = jnp.full((NUM_CONT_EMB, 1), -jnp.inf, jnp.float32)
    pos = jnp.full((NUM_CONT_EMB, BPAD - CONT_BINS - 1), jnp.inf, jnp.float32)
    bpad = jnp.concatenate([neg, cont_boundaries, pos], axis=1)

    b0 = cont_boundaries[:, 0]
    inv_step = (CONT_BINS - 1) / (cont_boundaries[:, -1] - b0)
    sparams = jnp.stack([
        jnp.full((L,), b0[0], jnp.float32),
        jnp.full((L,), b0[1], jnp.float32),
        jnp.full((L,), inv_step[0], jnp.float32),
        jnp.full((L,), inv_step[1], jnp.float32),
        jnp.tile(norm_mean.astype(jnp.float32), L // NUM_CONT_NORM),
        jnp.tile(norm_std.astype(jnp.float32), L // NUM_CONT_NORM),
    ])

    return _tower(cat_idx_t, hash_idx_t, cont_vals_t, norm_flat,
                  cat_tables, hash_tables, cont_tables, bpad, sparams)


# speedup vs baseline: 6.9565x; 6.9565x over previous
import functools

import jax
import jax.numpy as jnp
from jax import lax
from jax.experimental import pallas as pl
from jax.experimental.pallas import tpu as pltpu
from jax.experimental.pallas import tpu_sc as plsc

NUM_CAT = 3
CAT_VOCAB = 100000
NUM_HASH = 3
HASH_BINS = 100000
NUM_CONT_EMB = 2
CONT_BINS = 1000
NUM_CONT_NORM = 2
EMB = 32
BATCH = 16384
OUT_COLS = NUM_CAT * EMB + NUM_HASH * EMB + NUM_CONT_EMB * EMB + NUM_CONT_NORM

_info = plsc.get_sparse_core_info()
NC = _info.num_cores
NS = _info.num_subcores
L = _info.num_lanes
NW = NC * NS
CHUNK = BATCH // NW

BPAD = 1024


def _tower_body(cat_idx_t, hash_idx_t, cont_vals_t, norm_flat,
                cat_tab, hash_tab, cont_tab, bpad, sparams,
                out,
                idxb, rowsb, valsb, bndb, normb, prmb, sem):
    wid = lax.axis_index("s") * NC + lax.axis_index("c")
    base = wid * CHUNK

    pltpu.sync_copy(sparams, prmb)

    for t in range(NUM_CAT):
        pltpu.sync_copy(cat_idx_t.at[t, pl.ds(base, CHUNK)], idxb)
        pltpu.async_copy(cat_tab.at[t].at[idxb], rowsb, sem).wait()
        pltpu.sync_copy(rowsb, out.at[pl.ds(base, CHUNK), pl.ds(t * EMB, EMB)])
    for t in range(NUM_HASH):
        pltpu.sync_copy(hash_idx_t.at[t, pl.ds(base, CHUNK)], idxb)
        pltpu.async_copy(hash_tab.at[t].at[idxb], rowsb, sem).wait()
        pltpu.sync_copy(
            rowsb, out.at[pl.ds(base, CHUNK), pl.ds((NUM_CAT + t) * EMB, EMB)])

    for t in range(NUM_CONT_EMB):
        pltpu.sync_copy(cont_vals_t.at[t, pl.ds(base, CHUNK)],
                        valsb.at[pl.ds(0, CHUNK)])
        pltpu.sync_copy(bpad.at[t], bndb)
        b0 = prmb[t]
        ist = prmb[2 + t]
        for i in range(CHUNK // L):
            x = valsb[pl.ds(i * L, L)]
            est = (x - b0) * ist
            est = jnp.minimum(jnp.maximum(est, -2.0), float(CONT_BINS + 1))
            c = est.astype(jnp.int32) + 1
            c = jnp.minimum(jnp.maximum(c, 0), CONT_BINS)
            for _ in range(2):
                lo = plsc.load_gather(bndb, [c])
                hi = plsc.load_gather(bndb, [c + 1])
                c = c - (x < lo).astype(jnp.int32) + (x >= hi).astype(jnp.int32)
            idxb[pl.ds(i * L, L)] = c
        pltpu.async_copy(cont_tab.at[t].at[idxb], rowsb, sem).wait()
        pltpu.sync_copy(
            rowsb,
            out.at[pl.ds(base, CHUNK),
                   pl.ds((NUM_CAT + NUM_HASH + t) * EMB, EMB)])

    pltpu.sync_copy(norm_flat.at[pl.ds(base * NUM_CONT_NORM,
                                       CHUNK * NUM_CONT_NORM)], valsb)
    mean_pat = prmb[4]
    std_pat = prmb[5]
    lane = lax.iota(jnp.int32, L)
    col_idx = lane & 1
    for i in range(CHUNK * NUM_CONT_NORM // L):
        x = valsb[pl.ds(i * L, L)]
        y = (x - mean_pat) / std_pat
        row_idx = (lane + i * L) >> 1
        plsc.store_scatter(normb, [row_idx, col_idx], y)
    pltpu.sync_copy(
        normb,
        out.at[pl.ds(base, CHUNK),
               pl.ds((NUM_CAT + NUM_HASH + NUM_CONT_EMB) * EMB,
                     NUM_CONT_NORM)])


_tower = functools.partial(
    pl.kernel,
    mesh=plsc.VectorSubcoreMesh(core_axis_name="c", subcore_axis_name="s"),
    out_type=jax.ShapeDtypeStruct((BATCH, OUT_COLS), jnp.float32),
    scratch_types=[
        pltpu.VMEM((CHUNK,), jnp.int32),
        pltpu.VMEM((CHUNK, EMB), jnp.float32),
        pltpu.VMEM((CHUNK * NUM_CONT_NORM,), jnp.float32),
        pltpu.VMEM((BPAD,), jnp.float32),
        pltpu.VMEM((CHUNK, NUM_CONT_NORM), jnp.float32),
        pltpu.VMEM((6, L), jnp.float32),
        pltpu.SemaphoreType.DMA,
    ],
    compiler_params=pltpu.CompilerParams(
        use_tc_tiling_on_sc=False, needs_layout_passes=False),
)(_tower_body)


def kernel(cat_idx, hash_idx, cont_embed_vals, cont_norm_vals, cat_tables,
           hash_tables, cont_tables, cont_boundaries, norm_mean, norm_std):
    cat_idx_t = cat_idx.astype(jnp.int32).T
    hash_idx_t = hash_idx.astype(jnp.int32).T
    cont_vals_t = cont_embed_vals.T
    norm_flat = cont_norm_vals.reshape(-1)

    neg = jnp.full((NUM_CONT_EMB, 1), -jnp.inf, jnp.float32)
    pos = jnp.full((NUM_CONT_EMB, BPAD - CONT_BINS - 1), jnp.inf, jnp.float32)
    bpad = jnp.concatenate([neg, cont_boundaries, pos], axis=1)

    b0 = cont_boundaries[:, 0]
    inv_step = (CONT_BINS - 1) / (cont_boundaries[:, -1] - b0)
    sparams = jnp.stack([
        jnp.full((L,), b0[0], jnp.float32),
        jnp.full((L,), b0[1], jnp.float32),
        jnp.full((L,), inv_step[0], jnp.float32),
        jnp.full((L,), inv_step[1], jnp.float32),
        jnp.tile(norm_mean.astype(jnp.float32), L // NUM_CONT_NORM),
        jnp.tile(norm_std.astype(jnp.float32), L // NUM_CONT_NORM),
    ])

    return _tower(cat_idx_t, hash_idx_t, cont_vals_t, norm_flat,
                  cat_tables, hash_tables, cont_tables, bpad, sparams)
